# baseline (speedup 1.0000x reference)
import functools

import jax
import jax.numpy as jnp
from jax.experimental import pallas as pl


def _xw_kernel(x_ref, w_ref, o_ref):
    o_ref[...] = jnp.dot(x_ref[...], w_ref[...],
                         preferred_element_type=jnp.float32)


def _gc1_kernel(adj_ref, s1_ref, b1_ref, w2_ref, o_ref):
    h = jnp.dot(adj_ref[...], s1_ref[...], preferred_element_type=jnp.float32)
    h = jnp.maximum(h + b1_ref[...], 0.0)
    o_ref[...] = jnp.dot(h, w2_ref[...], preferred_element_type=jnp.float32)


def _gc2_kernel(adj_ref, s2_ref, b2_ref, o_ref):
    z = jnp.dot(adj_ref[...], s2_ref[...], preferred_element_type=jnp.float32)
    o_ref[...] = z + b2_ref[...]


def _decode_kernel(z_ref, o_ref, *, bm):
    i = pl.program_id(0)
    zi = z_ref[pl.ds(i * bm, bm), :]
    g = jax.lax.dot_general(zi, z_ref[...], (((1,), (1,)), ((), ())),
                            preferred_element_type=jnp.float32)
    o_ref[...] = jax.nn.sigmoid(g)


def kernel(x, adj, W1, b1, W2, b2):
    n, nfeat = x.shape
    nhid = W1.shape[1]
    nlat = W2.shape[1]
    b1r = b1.reshape(1, nhid)
    b2r = b2.reshape(1, nlat)

    bm0 = 1000 if n % 1000 == 0 else n
    s1 = pl.pallas_call(
        _xw_kernel,
        grid=(n // bm0,),
        in_specs=[
            pl.BlockSpec((bm0, nfeat), lambda i: (i, 0)),
            pl.BlockSpec((nfeat, nhid), lambda i: (0, 0)),
        ],
        out_specs=pl.BlockSpec((bm0, nhid), lambda i: (i, 0)),
        out_shape=jax.ShapeDtypeStruct((n, nhid), jnp.float32),
    )(x, W1)

    bm = 200 if n % 200 == 0 else n
    s2 = pl.pallas_call(
        _gc1_kernel,
        grid=(n // bm,),
        in_specs=[
            pl.BlockSpec((bm, n), lambda i: (i, 0)),
            pl.BlockSpec((n, nhid), lambda i: (0, 0)),
            pl.BlockSpec((1, nhid), lambda i: (0, 0)),
            pl.BlockSpec((nhid, nlat), lambda i: (0, 0)),
        ],
        out_specs=pl.BlockSpec((bm, nlat), lambda i: (i, 0)),
        out_shape=jax.ShapeDtypeStruct((n, nlat), jnp.float32),
    )(adj, s1, b1r, W2)

    z = pl.pallas_call(
        _gc2_kernel,
        grid=(n // bm,),
        in_specs=[
            pl.BlockSpec((bm, n), lambda i: (i, 0)),
            pl.BlockSpec((n, nlat), lambda i: (0, 0)),
            pl.BlockSpec((1, nlat), lambda i: (0, 0)),
        ],
        out_specs=pl.BlockSpec((bm, nlat), lambda i: (i, 0)),
        out_shape=jax.ShapeDtypeStruct((n, nlat), jnp.float32),
    )(adj, s2, b2r)

    bdm = 400 if n % 400 == 0 else n
    adj_rec = pl.pallas_call(
        functools.partial(_decode_kernel, bm=bdm),
        grid=(n // bdm,),
        in_specs=[
            pl.BlockSpec((n, nlat), lambda i: (0, 0)),
        ],
        out_specs=pl.BlockSpec((bdm, n), lambda i: (i, 0)),
        out_shape=jax.ShapeDtypeStruct((n, n), jnp.float32),
    )(z)

    return (adj_rec, z)


# speedup vs baseline: 1.0515x; 1.0515x over previous
import functools

import jax
import jax.numpy as jnp
from jax.experimental import pallas as pl
from jax.experimental.pallas import tpu as pltpu


def _gcn_kernel(adj_ref, x_ref, w1_ref, b1_ref, w2_ref, b2_ref,
                z_ref, s1_ref, s2_ref, *, bm):
    p = pl.program_id(0)
    i = pl.program_id(1)

    @pl.when((p == 0) & (i == 0))
    def _():
        s1_ref[...] = jnp.dot(x_ref[...], w1_ref[...],
                              preferred_element_type=jnp.float32)

    @pl.when(p == 0)
    def _():
        h = jnp.dot(adj_ref[...], s1_ref[...],
                    preferred_element_type=jnp.float32)
        h = jnp.maximum(h + b1_ref[...], 0.0)
        s2_ref[pl.ds(i * bm, bm), :] = jnp.dot(
            h, w2_ref[...], preferred_element_type=jnp.float32)

    @pl.when(p == 1)
    def _():
        z_ref[...] = jnp.dot(adj_ref[...], s2_ref[...],
                             preferred_element_type=jnp.float32) + b2_ref[...]


def _decode_kernel(z_ref, o_ref, *, bm):
    i = pl.program_id(0)
    zi = z_ref[pl.ds(i * bm, bm), :]
    g = jax.lax.dot_general(zi, z_ref[...], (((1,), (1,)), ((), ())),
                            preferred_element_type=jnp.float32)
    o_ref[...] = jax.nn.sigmoid(g)


def kernel(x, adj, W1, b1, W2, b2):
    n, nfeat = x.shape
    nhid = W1.shape[1]
    nlat = W2.shape[1]
    b1r = b1.reshape(1, nhid)
    b2r = b2.reshape(1, nlat)

    bm = 400 if n % 400 == 0 else n
    z = pl.pallas_call(
        functools.partial(_gcn_kernel, bm=bm),
        grid=(2, n // bm),
        in_specs=[
            pl.BlockSpec((bm, n), lambda p, i: (i, 0)),
            pl.BlockSpec((n, nfeat), lambda p, i: (0, 0)),
            pl.BlockSpec((nfeat, nhid), lambda p, i: (0, 0)),
            pl.BlockSpec((1, nhid), lambda p, i: (0, 0)),
            pl.BlockSpec((nhid, nlat), lambda p, i: (0, 0)),
            pl.BlockSpec((1, nlat), lambda p, i: (0, 0)),
        ],
        out_specs=pl.BlockSpec((bm, nlat), lambda p, i: (p * i, 0)),
        out_shape=jax.ShapeDtypeStruct((n, nlat), jnp.float32),
        scratch_shapes=[
            pltpu.VMEM((n, nhid), jnp.float32),
            pltpu.VMEM((n, nlat), jnp.float32),
        ],
    )(adj, x, W1, b1r, W2, b2r)

    bdm = 400 if n % 400 == 0 else n
    adj_rec = pl.pallas_call(
        functools.partial(_decode_kernel, bm=bdm),
        grid=(n // bdm,),
        in_specs=[
            pl.BlockSpec((n, nlat), lambda i: (0, 0)),
        ],
        out_specs=pl.BlockSpec((bdm, n), lambda i: (i, 0)),
        out_shape=jax.ShapeDtypeStruct((n, n), jnp.float32),
    )(z)

    return (adj_rec, z)
